# Initial kernel scaffold; baseline (speedup 1.0000x reference)
import jax
import jax.numpy as jnp
from jax.experimental import pallas as pl


def kernel(input_ids, orig_table, new_table):
    raise NotImplementedError("write your pallas kernel here")


# speedup vs baseline: 1.7806x; 1.7806x over previous
import functools

import jax
import jax.numpy as jnp
from jax import lax
from jax.experimental import pallas as pl
from jax.experimental.pallas import tpu as pltpu
from jax.experimental.pallas import tpu_sc as plsc

VOCAB = 32000
NUM_NEW = 1024
HIDDEN = 2048
BATCH = 4
SEQ = 8192
TOTAL = BATCH * SEQ

NUM_CORES = 2
NUM_SUBCORES = 16
NW = NUM_CORES * NUM_SUBCORES
PER_W = TOTAL // NW
C = 16
NCHUNK = PER_W // C

_mesh = plsc.VectorSubcoreMesh(core_axis_name="c", subcore_axis_name="s")


@functools.partial(
    pl.kernel,
    mesh=_mesh,
    out_type=jax.ShapeDtypeStruct((TOTAL, HIDDEN), jnp.float32),
    scratch_types=[
        pltpu.VMEM((C,), jnp.int32),
        pltpu.VMEM((C,), jnp.int32),
        pltpu.VMEM((C, HIDDEN), jnp.float32),
        pltpu.SemaphoreType.DMA,
    ],
)
def _encode(ids_hbm, orig_hbm, new_hbm, out_hbm, idx_v, gidx_v, rows_v, sem):
    wid = lax.axis_index("s") * NUM_CORES + lax.axis_index("c")
    base = wid * PER_W

    def chunk_body(ci, carry):
        gbase = base + ci * C
        pltpu.sync_copy(ids_hbm.at[pl.ds(gbase, C)], idx_v)
        for g in range(C // 16):
            v = idx_v[pl.ds(g * 16, 16)]
            gidx_v[pl.ds(g * 16, 16)] = jnp.minimum(v, VOCAB - 1)
        pltpu.async_copy(orig_hbm.at[gidx_v], rows_v, sem).wait()

        for g in range(C // 16):
            v = idx_v[pl.ds(g * 16, 16)]
            for lane in range(16):
                tid = v[lane]

                @pl.when(tid >= VOCAB)
                def _():
                    pltpu.sync_copy(
                        new_hbm.at[pl.ds(tid - VOCAB, 1)],
                        rows_v.at[pl.ds(g * 16 + lane, 1)],
                    )
        pltpu.sync_copy(rows_v, out_hbm.at[pl.ds(gbase, C)])
        return carry

    lax.fori_loop(0, NCHUNK, chunk_body, 0)


def kernel(input_ids, orig_table, new_table):
    ids = input_ids.reshape(TOTAL).astype(jnp.int32)
    out = _encode(ids, orig_table, new_table)
    return out.reshape(BATCH, SEQ, HIDDEN)


# speedup vs baseline: 2.3390x; 1.3136x over previous
import functools

import jax
import jax.numpy as jnp
from jax import lax
from jax.experimental import pallas as pl
from jax.experimental.pallas import tpu as pltpu
from jax.experimental.pallas import tpu_sc as plsc

VOCAB = 32000
NUM_NEW = 1024
HIDDEN = 2048
BATCH = 4
SEQ = 8192
TOTAL = BATCH * SEQ

NUM_CORES = 2
NUM_SUBCORES = 16
NW = NUM_CORES * NUM_SUBCORES
PER_W = TOTAL // NW
C = 16
NCHUNK = PER_W // C
NPAIR = NCHUNK // 2

_mesh = plsc.VectorSubcoreMesh(core_axis_name="c", subcore_axis_name="s")


@functools.partial(
    pl.kernel,
    mesh=_mesh,
    out_type=jax.ShapeDtypeStruct((TOTAL, HIDDEN), jnp.float32),
    scratch_types=[
        pltpu.VMEM((PER_W,), jnp.int32),
        pltpu.VMEM((PER_W,), jnp.int32),
        pltpu.VMEM((C, HIDDEN), jnp.float32),
        pltpu.VMEM((C, HIDDEN), jnp.float32),
        pltpu.SemaphoreType.DMA,
        pltpu.SemaphoreType.DMA,
        pltpu.SemaphoreType.DMA,
        pltpu.SemaphoreType.DMA,
        pltpu.SemaphoreType.DMA,
    ],
)
def _encode(ids_hbm, orig_hbm, new_hbm, out_hbm,
            idx_all, gidx_all, rows0, rows1,
            gsem0, gsem1, wsem0, wsem1, psem):
    rows = (rows0, rows1)
    gsem = (gsem0, gsem1)
    wsem = (wsem0, wsem1)
    wid = lax.axis_index("s") * NUM_CORES + lax.axis_index("c")
    base = wid * PER_W

    pltpu.sync_copy(ids_hbm.at[pl.ds(base, PER_W)], idx_all)

    def clamp_grp(g, carry):
        v = idx_all[pl.ds(g * 16, 16)]
        gidx_all[pl.ds(g * 16, 16)] = jnp.minimum(v, VOCAB - 1)
        return carry

    lax.fori_loop(0, PER_W // 16, clamp_grp, 0)

    def start_gather(ci, b):
        pltpu.async_copy(
            orig_hbm.at[gidx_all.at[pl.ds(ci * C, C)]], rows[b], gsem[b])

    def wait_gather(ci, b):
        pltpu.make_async_copy(
            orig_hbm.at[gidx_all.at[pl.ds(ci * C, C)]], rows[b],
            gsem[b]).wait()

    def start_write(ci, b):
        pltpu.async_copy(rows[b], out_hbm.at[pl.ds(base + ci * C, C)],
                         wsem[b])

    def wait_write(ci, b):
        pltpu.make_async_copy(rows[b], out_hbm.at[pl.ds(base + ci * C, C)],
                              wsem[b]).wait()

    start_gather(0, 0)

    def pair_body(pair, carry):
        for b in range(2):
            ci = pair * 2 + b
            nb = 1 - b
            nci = ci + 1

            @pl.when(jnp.logical_and(nci < NCHUNK, ci >= 1))
            def _():
                wait_write(ci - 1, nb)

            @pl.when(nci < NCHUNK)
            def _():
                start_gather(nci, nb)

            wait_gather(ci, b)

            n = jnp.int32(0)
            for g in range(C // 16):
                v = idx_all[pl.ds(ci * C + g * 16, 16)]
                for lane in range(16):
                    tid = v[lane]
                    n = n + (tid >= VOCAB).astype(jnp.int32)

                    @pl.when(tid >= VOCAB)
                    def _():
                        pltpu.async_copy(
                            new_hbm.at[pl.ds(tid - VOCAB, 1)],
                            rows[b].at[pl.ds(g * 16 + lane, 1)],
                            psem)

            def drain(i, carry2):
                pltpu.make_async_copy(
                    new_hbm.at[pl.ds(0, 1)], rows[b].at[pl.ds(0, 1)],
                    psem).wait()
                return carry2

            lax.fori_loop(0, n, drain, 0)

            start_write(ci, b)
        return carry

    lax.fori_loop(0, NPAIR, pair_body, 0)
    wait_write(NCHUNK - 2, 0)
    wait_write(NCHUNK - 1, 1)


def kernel(input_ids, orig_table, new_table):
    ids = input_ids.reshape(TOTAL).astype(jnp.int32)
    out = _encode(ids, orig_table, new_table)
    return out.reshape(BATCH, SEQ, HIDDEN)
